# baseline (speedup 1.0000x reference)
import jax
import jax.numpy as jnp
from jax.experimental import pallas as pl
from jax.experimental.pallas import tpu as pltpu


def kernel(x_nchw, w1, b1, w2, b2):
    n, c, h, w = x_nchw.shape
    hidden = w1.shape[1]
    inv_s = 1.0 / float(h * w)

    b1c = b1.reshape(hidden, 1)
    b2c = b2.reshape(1, 1)

    def _body(x_ref, w1_ref, b1_ref, w2_ref, b2_ref, o_ref):
        tile = x_ref[0]
        s1 = jnp.sum(tile, axis=1)
        pooled = jnp.sum(s1, axis=1, keepdims=True) * inv_s
        hid = jax.lax.dot_general(
            w1_ref[...], pooled, (((0,), (0,)), ((), ())),
            preferred_element_type=jnp.float32)
        hid = jnp.maximum(hid + b1_ref[...], 0.0)
        logit = jax.lax.dot_general(
            w2_ref[...], hid, (((0,), (0,)), ((), ())),
            preferred_element_type=jnp.float32) + b2_ref[...]
        o_ref[...] = jax.nn.sigmoid(logit).reshape(1, 1, 1)

    out = pl.pallas_call(
        _body,
        out_shape=jax.ShapeDtypeStruct((n, 1, 1), jnp.float32),
        grid=(n,),
        in_specs=[
            pl.BlockSpec((1, c, h, w), lambda i: (i, 0, 0, 0)),
            pl.BlockSpec((c, hidden), lambda i: (0, 0)),
            pl.BlockSpec((hidden, 1), lambda i: (0, 0)),
            pl.BlockSpec((hidden, 1), lambda i: (0, 0)),
            pl.BlockSpec((1, 1), lambda i: (0, 0)),
        ],
        out_specs=pl.BlockSpec((1, 1, 1), lambda i: (i, 0, 0)),
        compiler_params=pltpu.CompilerParams(
            dimension_semantics=("parallel",),
            vmem_limit_bytes=64 * 1024 * 1024,
        ),
    )(x_nchw, w1, b1c, w2, b2c)
    return out.reshape(n, 1)


# speedup vs baseline: 4.8983x; 4.8983x over previous
import jax
import jax.numpy as jnp
from jax.experimental import pallas as pl
from jax.experimental.pallas import tpu as pltpu


def kernel(x_nchw, w1, b1, w2, b2):
    n, c, h, w = x_nchw.shape
    s = h * w
    hidden = w1.shape[1]
    inv_s = 1.0 / float(s)

    x_sc = jnp.transpose(x_nchw, (0, 2, 3, 1)).reshape(n, s, c)
    b1r = b1.reshape(1, hidden)
    b2r = b2.reshape(1, 1)

    def _body(x_ref, w1_ref, b1_ref, w2_ref, b2_ref, o_ref):
        tile = x_ref[0]
        pooled = jnp.sum(tile, axis=0, keepdims=True) * inv_s
        hid = jnp.dot(pooled, w1_ref[...],
                      preferred_element_type=jnp.float32)
        hid = jnp.maximum(hid + b1_ref[...], 0.0)
        logit = jnp.dot(hid, w2_ref[...],
                        preferred_element_type=jnp.float32) + b2_ref[...]
        o_ref[...] = jax.nn.sigmoid(logit).reshape(1, 1, 1)

    out = pl.pallas_call(
        _body,
        out_shape=jax.ShapeDtypeStruct((n, 1, 1), jnp.float32),
        grid=(n,),
        in_specs=[
            pl.BlockSpec((1, s, c), lambda i: (i, 0, 0)),
            pl.BlockSpec((c, hidden), lambda i: (0, 0)),
            pl.BlockSpec((1, hidden), lambda i: (0, 0)),
            pl.BlockSpec((hidden, 1), lambda i: (0, 0)),
            pl.BlockSpec((1, 1), lambda i: (0, 0)),
        ],
        out_specs=pl.BlockSpec((1, 1, 1), lambda i: (i, 0, 0)),
        compiler_params=pltpu.CompilerParams(
            dimension_semantics=("parallel",),
            vmem_limit_bytes=64 * 1024 * 1024,
        ),
    )(x_sc, w1, b1r, w2, b2r)
    return out.reshape(n, 1)


# speedup vs baseline: 6.9661x; 1.4221x over previous
import jax
import jax.numpy as jnp
from jax.experimental import pallas as pl
from jax.experimental.pallas import tpu as pltpu


def kernel(x_nchw, w1, b1, w2, b2):
    n, c, h, w = x_nchw.shape
    s = h * w
    hidden = w1.shape[1]
    inv_s = 1.0 / float(s)

    x_sc = jnp.transpose(x_nchw, (0, 2, 3, 1)).reshape(n, s, c)
    b1r = b1.reshape(1, hidden)
    b2r = b2.reshape(1, 1)

    bs = 4 if n % 4 == 0 else 1

    def _body(x_ref, w1_ref, b1_ref, w2_ref, b2_ref, o_ref):
        tile = x_ref[...]
        pooled = jnp.sum(tile, axis=1) * inv_s
        hid = jnp.dot(pooled, w1_ref[...],
                      preferred_element_type=jnp.float32)
        hid = jnp.maximum(hid + b1_ref[...], 0.0)
        logit = jnp.dot(hid, w2_ref[...],
                        preferred_element_type=jnp.float32) + b2_ref[...]
        o_ref[...] = jax.nn.sigmoid(logit).reshape(bs, 1, 1)

    out = pl.pallas_call(
        _body,
        out_shape=jax.ShapeDtypeStruct((n, 1, 1), jnp.float32),
        grid=(n // bs,),
        in_specs=[
            pl.BlockSpec((bs, s, c), lambda i: (i, 0, 0)),
            pl.BlockSpec((c, hidden), lambda i: (0, 0)),
            pl.BlockSpec((1, hidden), lambda i: (0, 0)),
            pl.BlockSpec((hidden, 1), lambda i: (0, 0)),
            pl.BlockSpec((1, 1), lambda i: (0, 0)),
        ],
        out_specs=pl.BlockSpec((bs, 1, 1), lambda i: (i, 0, 0)),
        compiler_params=pltpu.CompilerParams(
            dimension_semantics=("parallel",),
            vmem_limit_bytes=64 * 1024 * 1024,
        ),
    )(x_sc, w1, b1r, w2, b2r)
    return out.reshape(n, 1)


# speedup vs baseline: 7.0628x; 1.0139x over previous
import jax
import jax.numpy as jnp
from jax.experimental import pallas as pl
from jax.experimental.pallas import tpu as pltpu


def kernel(x_nchw, w1, b1, w2, b2):
    n, c, h, w = x_nchw.shape
    s = h * w
    hidden = w1.shape[1]
    inv_s = 1.0 / float(s)

    x_sc = jnp.transpose(x_nchw, (0, 2, 3, 1)).reshape(n, s, c)
    b1r = b1.reshape(1, hidden)
    w2r = w2.reshape(1, hidden)
    b2r = b2.reshape(1, 1)

    bs = 8 if n % 8 == 0 else (4 if n % 4 == 0 else 1)

    def _body(x_ref, w1_ref, b1_ref, w2_ref, b2_ref, o_ref):
        tile = x_ref[...]
        pooled = jnp.sum(tile, axis=1) * inv_s
        hid = jnp.dot(pooled, w1_ref[...],
                      preferred_element_type=jnp.float32)
        hid = jnp.maximum(hid + b1_ref[...], 0.0)
        logit = jnp.sum(hid * w2_ref[...], axis=1,
                        keepdims=True) + b2_ref[...]
        o_ref[...] = jax.nn.sigmoid(logit)

    return pl.pallas_call(
        _body,
        out_shape=jax.ShapeDtypeStruct((n, 1), jnp.float32),
        grid=(n // bs,),
        in_specs=[
            pl.BlockSpec((bs, s, c), lambda i: (i, 0, 0)),
            pl.BlockSpec((c, hidden), lambda i: (0, 0)),
            pl.BlockSpec((1, hidden), lambda i: (0, 0)),
            pl.BlockSpec((1, hidden), lambda i: (0, 0)),
            pl.BlockSpec((1, 1), lambda i: (0, 0)),
        ],
        out_specs=pl.BlockSpec((bs, 1), lambda i: (i, 0)),
        compiler_params=pltpu.CompilerParams(
            dimension_semantics=("parallel",),
            vmem_limit_bytes=58 * 1024 * 1024,
        ),
    )(x_sc, w1, b1r, w2r, b2r)
